# baseline (speedup 1.0000x reference)
import jax
import jax.numpy as jnp
from jax import lax
from jax.experimental import pallas as pl
from jax.experimental.pallas import tpu as pltpu
from jax.experimental.pallas import tpu_sc as plsc

BATCH = 16384
LATENT_DIM = 16
NUM_WORKERS = 32
PAIRS_PER_WORKER = BATCH // NUM_WORKERS
CHUNK = 128
CHUNKS_PER_WORKER = PAIRS_PER_WORKER // CHUNK
IDS_COLS = 128
IDS_ROWS = BATCH // IDS_COLS


def _sc_body(bill_ids, leg_ids, global_bias, leg_bias, bill_bias,
             leg_emb, bill_emb, out_hbm,
             bidx_v, lidx_v, bill_rows, leg_rows, bb_v, lb_v, gb_v, out_v,
             sem):
    nc = lax.axis_index("c")
    ns = lax.axis_index("s")
    wid = ns * 2 + nc
    idx_row0 = wid * (PAIRS_PER_WORKER // IDS_COLS)

    pltpu.sync_copy(bill_ids.at[pl.ds(idx_row0, CHUNKS_PER_WORKER)], bidx_v)
    pltpu.sync_copy(leg_ids.at[pl.ds(idx_row0, CHUNKS_PER_WORKER)], lidx_v)
    pltpu.sync_copy(global_bias, gb_v)

    copies = []
    for j in range(CHUNKS_PER_WORKER):
        dst = pl.ds(j * CHUNK, CHUNK)
        copies.append(pltpu.async_copy(
            bill_emb.at[bidx_v.at[j]], bill_rows.at[dst], sem))
        copies.append(pltpu.async_copy(
            leg_emb.at[lidx_v.at[j]], leg_rows.at[dst], sem))
        copies.append(pltpu.async_copy(
            bill_bias.at[bidx_v.at[j]], bb_v.at[dst], sem))
        copies.append(pltpu.async_copy(
            leg_bias.at[lidx_v.at[j]], lb_v.at[dst], sem))
    for c in copies:
        c.wait()

    gb = gb_v[...]
    lane = lax.iota(jnp.int32, 16)

    def group(g, _):
        row = g * 16 + lane
        acc = jnp.zeros((16,), jnp.float32)
        for d in range(LATENT_DIM):
            col = jnp.full((16,), d, jnp.int32)
            acc = acc + (plsc.load_gather(bill_rows, [row, col])
                         * plsc.load_gather(leg_rows, [row, col]))
        x = gb + bb_v[pl.ds(g * 16, 16)] + lb_v[pl.ds(g * 16, 16)] + acc
        out_v[pl.ds(g * 16, 16)] = 1.0 / (1.0 + jnp.exp(-x))
        return 0

    lax.fori_loop(0, PAIRS_PER_WORKER // 16, group, 0)

    pltpu.sync_copy(out_v, out_hbm.at[pl.ds(wid * PAIRS_PER_WORKER,
                                            PAIRS_PER_WORKER)])


@jax.jit
def _predict(bill_ids, leg_ids, global_bias, leg_bias, bill_bias,
             leg_emb, bill_emb):
    mesh = plsc.VectorSubcoreMesh(core_axis_name="c", subcore_axis_name="s")
    k = pl.kernel(
        _sc_body,
        out_type=jax.ShapeDtypeStruct((BATCH,), jnp.float32),
        mesh=mesh,
        compiler_params=pltpu.CompilerParams(needs_layout_passes=False,
                                             use_tc_tiling_on_sc=False),
        scratch_types=[
            pltpu.VMEM((CHUNKS_PER_WORKER, CHUNK), jnp.int32),
            pltpu.VMEM((CHUNKS_PER_WORKER, CHUNK), jnp.int32),
            pltpu.VMEM((PAIRS_PER_WORKER, LATENT_DIM), jnp.float32),
            pltpu.VMEM((PAIRS_PER_WORKER, LATENT_DIM), jnp.float32),
            pltpu.VMEM((PAIRS_PER_WORKER,), jnp.float32),
            pltpu.VMEM((PAIRS_PER_WORKER,), jnp.float32),
            pltpu.VMEM((16,), jnp.float32),
            pltpu.VMEM((PAIRS_PER_WORKER,), jnp.float32),
            pltpu.SemaphoreType.DMA,
        ],
    )
    return k(bill_ids, leg_ids, global_bias, leg_bias, bill_bias,
             leg_emb, bill_emb)


def kernel(bill_ids, legislator_ids, global_bias, legislator_bias, bill_bias,
           legislator_embedding, bill_embedding):
    bids = jnp.reshape(bill_ids.astype(jnp.int32), (IDS_ROWS, IDS_COLS))
    lids = jnp.reshape(legislator_ids.astype(jnp.int32), (IDS_ROWS, IDS_COLS))
    leg_b = jnp.reshape(legislator_bias, (-1,))
    bill_b = jnp.reshape(bill_bias, (-1,))
    gb = jnp.broadcast_to(jnp.reshape(global_bias, (1,)), (16,))
    return _predict(bids, lids, gb, leg_b, bill_b,
                    legislator_embedding, bill_embedding)
